# Initial kernel scaffold; baseline (speedup 1.0000x reference)
import jax
import jax.numpy as jnp
from jax.experimental import pallas as pl


def kernel(feat0, params, adj, type_mask):
    raise NotImplementedError("write your pallas kernel here")


# speedup vs baseline: 16.9108x; 16.9108x over previous
import functools

import jax
import jax.numpy as jnp
from jax import lax
from jax.experimental import pallas as pl
from jax.experimental.pallas import tpu as pltpu
from jax.experimental.pallas import tpu_sc as plsc

N = 10000
E = 320000
DF = 128
HID = 256
HEADS = 8
DH = 32
OUT = 128
NC = 8
M = 2
ATT = 128
SLICE = 4019

HALF = HID // 2
EXTW = HALF + 16
EDW = 16
NSUB = 16
EPT = E // NSUB
CHUNK = 80
NCHUNK = EPT // CHUNK
NP = 10240
RPT = NP // NSUB

BLK = 1000
GRID = N // BLK
HI = lax.Precision.HIGHEST


def _elu(x):
    return jnp.where(x > 0, x, jnp.exp(x) - 1.0)


def _gat_edge_body(hdA, hdB, edt, srcs, dsts, gvec, zinit,
                   accA, accB,
                   acc, srcb, dstb, rows, edr, stag, gv, sem):
    c = lax.axis_index("c")
    s = lax.axis_index("s")
    pltpu.sync_copy(zinit.at[pl.ds(s * RPT, RPT)], acc.at[pl.ds(s * RPT, RPT)])
    pltpu.sync_copy(gvec, gv)
    plsc.subcore_barrier()
    g = gv[...]
    iot = lax.iota(jnp.int32, 16)
    hbase = c * 4

    def chunk(k, carry):
        base = s * EPT + k * CHUNK
        pltpu.sync_copy(srcs.at[pl.ds(base, CHUNK)], srcb)
        pltpu.sync_copy(dsts.at[pl.ds(base, CHUNK)], dstb)

        @pl.when(c == 0)
        def _():
            pltpu.async_copy(hdA.at[srcb], rows, sem).wait()

        @pl.when(c == 1)
        def _():
            pltpu.async_copy(hdB.at[srcb], rows, sem).wait()

        pltpu.async_copy(edt.at[dstb], edr, sem).wait()

        def edge(i, carry2):
            i16 = jnp.full((16,), i, jnp.int32)
            esv = plsc.load_gather(rows, [i16, HALF + iot])
            edv = plsc.load_gather(edr, [i16, iot])
            e = esv + edv
            e = jnp.where(e < 0.0, 0.2 * e, e)
            ex = jnp.exp(e - g)
            plsc.store_scatter(stag, [i16, HALF + iot], ex)
            for j in range(HALF // 16):
                fv = plsc.load_gather(rows, [i16, j * 16 + iot])
                hsel = jnp.full((16,), j // 2, jnp.int32) + hbase
                bc = lax.gather(
                    ex, hsel[:, None],
                    dimension_numbers=lax.GatherDimensionNumbers(
                        offset_dims=(), collapsed_slice_dims=(0,),
                        start_index_map=(0,)),
                    slice_sizes=(1,),
                    mode=lax.GatherScatterMode.PROMISE_IN_BOUNDS)
                plsc.store_scatter(stag, [i16, j * 16 + iot], fv * bc)
            return carry2

        lax.fori_loop(0, CHUNK, edge, 0)
        pltpu.sync_copy(stag, acc.at[dstb], add=True)
        return carry

    lax.fori_loop(0, NCHUNK, chunk, 0)
    plsc.subcore_barrier()

    @pl.when(c == 0)
    def _():
        pltpu.sync_copy(acc.at[pl.ds(s * RPT, RPT)], accA.at[pl.ds(s * RPT, RPT)])

    @pl.when(c == 1)
    def _():
        pltpu.sync_copy(acc.at[pl.ds(s * RPT, RPT)], accB.at[pl.ds(s * RPT, RPT)])


_gat_edge = pl.kernel(
    _gat_edge_body,
    out_type=(jax.ShapeDtypeStruct((NP, EXTW), jnp.float32),
              jax.ShapeDtypeStruct((NP, EXTW), jnp.float32)),
    mesh=plsc.VectorSubcoreMesh(core_axis_name="c", subcore_axis_name="s"),
    compiler_params=pltpu.CompilerParams(needs_layout_passes=False, use_tc_tiling_on_sc=False),
    scratch_types=[
        pltpu.VMEM_SHARED((NP, EXTW), jnp.float32),
        pltpu.VMEM((CHUNK,), jnp.int32),
        pltpu.VMEM((CHUNK,), jnp.int32),
        pltpu.VMEM((CHUNK, EXTW), jnp.float32),
        pltpu.VMEM((CHUNK, EDW), jnp.float32),
        pltpu.VMEM((CHUNK, EXTW), jnp.float32),
        pltpu.VMEM((16,), jnp.float32),
        pltpu.SemaphoreType.DMA,
    ],
)


def _agg_edge_body(hA, hB, srcs, dsts, zinit,
                   aggA, aggB,
                   acc, srcb, dstb, rows, sem):
    c = lax.axis_index("c")
    s = lax.axis_index("s")
    pltpu.sync_copy(zinit.at[pl.ds(s * RPT, RPT)], acc.at[pl.ds(s * RPT, RPT)])
    plsc.subcore_barrier()

    def chunk(k, carry):
        base = s * EPT + k * CHUNK
        pltpu.sync_copy(srcs.at[pl.ds(base, CHUNK)], srcb)
        pltpu.sync_copy(dsts.at[pl.ds(base, CHUNK)], dstb)

        @pl.when(c == 0)
        def _():
            pltpu.async_copy(hA.at[srcb], rows, sem).wait()

        @pl.when(c == 1)
        def _():
            pltpu.async_copy(hB.at[srcb], rows, sem).wait()

        pltpu.sync_copy(rows, acc.at[dstb], add=True)
        return carry

    lax.fori_loop(0, NCHUNK, chunk, 0)
    plsc.subcore_barrier()

    @pl.when(c == 0)
    def _():
        pltpu.sync_copy(acc.at[pl.ds(s * RPT, RPT)], aggA.at[pl.ds(s * RPT, RPT)])

    @pl.when(c == 1)
    def _():
        pltpu.sync_copy(acc.at[pl.ds(s * RPT, RPT)], aggB.at[pl.ds(s * RPT, RPT)])


_agg_edge = pl.kernel(
    _agg_edge_body,
    out_type=(jax.ShapeDtypeStruct((NP, HALF), jnp.float32),
              jax.ShapeDtypeStruct((NP, HALF), jnp.float32)),
    mesh=plsc.VectorSubcoreMesh(core_axis_name="c", subcore_axis_name="s"),
    compiler_params=pltpu.CompilerParams(needs_layout_passes=False, use_tc_tiling_on_sc=False),
    scratch_types=[
        pltpu.VMEM_SHARED((NP, HALF), jnp.float32),
        pltpu.VMEM((CHUNK,), jnp.int32),
        pltpu.VMEM((CHUNK,), jnp.int32),
        pltpu.VMEM((CHUNK, HALF), jnp.float32),
        pltpu.SemaphoreType.DMA,
    ],
)


def _row_spec(w):
    return pl.BlockSpec((BLK, w), lambda i: (i, 0))


def _full_spec(h, w):
    return pl.BlockSpec((h, w), lambda i: (0, 0))


def _t1_body(x_r, wfc_r, bfc_r, w1_r, as_r, ad_r,
             fs_r, hd_r, es_r, ed_r, gm_r):
    i = pl.program_id(0)
    fs = jnp.dot(x_r[...], wfc_r[...], precision=HI,
                 preferred_element_type=jnp.float32) + bfc_r[...]
    hd = jnp.dot(fs, w1_r[...], precision=HI,
                 preferred_element_type=jnp.float32)
    es = jnp.dot(hd, as_r[...], precision=HI,
                 preferred_element_type=jnp.float32)
    ed = jnp.dot(hd, ad_r[...], precision=HI,
                 preferred_element_type=jnp.float32)
    fs_r[...] = fs
    hd_r[...] = hd
    es_r[...] = es
    ed_r[...] = ed
    new = jnp.concatenate([jnp.max(es, axis=0, keepdims=True),
                           jnp.max(ed, axis=0, keepdims=True),
                           jnp.full((6, HEADS), -1e30, jnp.float32)], axis=0)

    @pl.when(i == 0)
    def _():
        gm_r[...] = new

    @pl.when(i > 0)
    def _():
        gm_r[...] = jnp.maximum(gm_r[...], new)


def _stage1(feat0, wfc, bfc, w1, a_s, a_d):
    return pl.pallas_call(
        _t1_body,
        grid=(GRID,),
        in_specs=[_row_spec(DF), _full_spec(DF, HID), _full_spec(1, HID),
                  _full_spec(HID, HID), _full_spec(HID, HEADS),
                  _full_spec(HID, HEADS)],
        out_specs=[_row_spec(HID), _row_spec(HID), _row_spec(HEADS),
                   _row_spec(HEADS), _full_spec(8, HEADS)],
        out_shape=[jax.ShapeDtypeStruct((N, HID), jnp.float32),
                   jax.ShapeDtypeStruct((N, HID), jnp.float32),
                   jax.ShapeDtypeStruct((N, HEADS), jnp.float32),
                   jax.ShapeDtypeStruct((N, HEADS), jnp.float32),
                   jax.ShapeDtypeStruct((8, HEADS), jnp.float32)],
    )(feat0, wfc, bfc, w1, a_s, a_d)


def _t2_body(aA_r, aB_r, exp_r, w2_r, as_r, ad_r,
             hd_r, es_r, ed_r, gm_r):
    i = pl.program_id(0)
    a = aA_r[...]
    b = aB_r[...]
    den = a[:, HALF:HALF + HEADS] + 1e-9
    denx = jnp.dot(den, exp_r[...], precision=HI,
                   preferred_element_type=jnp.float32)
    num = jnp.concatenate([a[:, :HALF], b[:, :HALF]], axis=1)
    h1 = _elu(num / denx)
    hd = jnp.dot(h1, w2_r[...], precision=HI,
                 preferred_element_type=jnp.float32)
    es = jnp.dot(hd, as_r[...], precision=HI,
                 preferred_element_type=jnp.float32)
    ed = jnp.dot(hd, ad_r[...], precision=HI,
                 preferred_element_type=jnp.float32)
    hd_r[...] = hd
    es_r[...] = es
    ed_r[...] = ed
    new = jnp.concatenate([jnp.max(es, axis=0, keepdims=True),
                           jnp.max(ed, axis=0, keepdims=True),
                           jnp.full((6, HEADS), -1e30, jnp.float32)], axis=0)

    @pl.when(i == 0)
    def _():
        gm_r[...] = new

    @pl.when(i > 0)
    def _():
        gm_r[...] = jnp.maximum(gm_r[...], new)


def _stage2(accA, accB, expand, w2, a_s, a_d):
    return pl.pallas_call(
        _t2_body,
        grid=(GRID,),
        in_specs=[_row_spec(EXTW), _row_spec(EXTW), _full_spec(HEADS, HID),
                  _full_spec(HID, HID), _full_spec(HID, HEADS),
                  _full_spec(HID, HEADS)],
        out_specs=[_row_spec(HID), _row_spec(HEADS), _row_spec(HEADS),
                   _full_spec(8, HEADS)],
        out_shape=[jax.ShapeDtypeStruct((N, HID), jnp.float32),
                   jax.ShapeDtypeStruct((N, HEADS), jnp.float32),
                   jax.ShapeDtypeStruct((N, HEADS), jnp.float32),
                   jax.ShapeDtypeStruct((8, HEADS), jnp.float32)],
    )(accA, accB, expand, w2, a_s, a_d)


def _t3_body(aA_r, aB_r, exp_r, fs_r, hA_r, hB_r, deg_r):
    i = pl.program_id(0)
    a = aA_r[...]
    b = aB_r[...]
    den = a[:, HALF:HALF + HEADS] + 1e-9
    denx = jnp.dot(den, exp_r[...], precision=HI,
                   preferred_element_type=jnp.float32)
    num = jnp.concatenate([a[:, :HALF], b[:, :HALF]], axis=1)
    h2 = num / denx
    ridx = lax.broadcasted_iota(jnp.int32, (BLK, 1), 0) + i * BLK
    h2 = jnp.where(ridx < SLICE, fs_r[...], h2)
    hA_r[...] = h2[:, :HALF]
    hB_r[...] = h2[:, HALF:]
    deg_r[...] = a[:, HALF + HEADS:]


def _stage3(accA, accB, expand, fs):
    return pl.pallas_call(
        _t3_body,
        grid=(GRID,),
        in_specs=[_row_spec(EXTW), _row_spec(EXTW), _full_spec(HEADS, HID),
                  _row_spec(HID)],
        out_specs=[_row_spec(HALF), _row_spec(HALF), _row_spec(8)],
        out_shape=[jax.ShapeDtypeStruct((N, HALF), jnp.float32),
                   jax.ShapeDtypeStruct((N, HALF), jnp.float32),
                   jax.ShapeDtypeStruct((N, 8), jnp.float32)],
    )(accA, accB, expand, fs)


def _t4_body(hA_r, hB_r, deg_r, w0_r, w1_r, b0_r, b1_r, wq_r, bq_r, q_r,
             z0_r, z1_r, ss_r):
    i = pl.program_id(0)
    deg = jnp.maximum(deg_r[...][:, 0:1], 1.0)
    aggv = jnp.concatenate([hA_r[...], hB_r[...]], axis=1) / deg
    z0 = _elu(jnp.dot(aggv, w0_r[...], precision=HI,
                      preferred_element_type=jnp.float32) + b0_r[...])
    z1 = _elu(jnp.dot(aggv, w1_r[...], precision=HI,
                      preferred_element_type=jnp.float32) + b1_r[...])
    z0_r[...] = z0
    z1_r[...] = z1
    t0 = jnp.tanh(jnp.dot(z0, wq_r[...], precision=HI,
                          preferred_element_type=jnp.float32) + bq_r[...])
    t1 = jnp.tanh(jnp.dot(z1, wq_r[...], precision=HI,
                          preferred_element_type=jnp.float32) + bq_r[...])
    s0 = jnp.dot(t0, q_r[...], precision=HI,
                 preferred_element_type=jnp.float32)
    s1 = jnp.dot(t1, q_r[...], precision=HI,
                 preferred_element_type=jnp.float32)
    new = jnp.concatenate([jnp.sum(s0, axis=0, keepdims=True),
                           jnp.sum(s1, axis=0, keepdims=True),
                           jnp.zeros((6, 8), jnp.float32)], axis=0)

    @pl.when(i == 0)
    def _():
        ss_r[...] = new

    @pl.when(i > 0)
    def _():
        ss_r[...] = ss_r[...] + new


def _stage4(hA, hB, degc, w0, w1, b0, b1, wq, bq, qv):
    return pl.pallas_call(
        _t4_body,
        grid=(GRID,),
        in_specs=[_row_spec(HALF), _row_spec(HALF), _row_spec(8),
                  _full_spec(HID, OUT * HEADS), _full_spec(HID, OUT * HEADS),
                  _full_spec(1, OUT * HEADS), _full_spec(1, OUT * HEADS),
                  _full_spec(OUT * HEADS, ATT), _full_spec(1, ATT),
                  _full_spec(ATT, 8)],
        out_specs=[_row_spec(OUT * HEADS), _row_spec(OUT * HEADS),
                   _full_spec(8, 8)],
        out_shape=[jax.ShapeDtypeStruct((N, OUT * HEADS), jnp.float32),
                   jax.ShapeDtypeStruct((N, OUT * HEADS), jnp.float32),
                   jax.ShapeDtypeStruct((8, 8), jnp.float32)],
    )(hA, hB, degc, w0, w1, b0, b1, wq, bq, qv)


def _t5_body(z0_r, z1_r, beta_r, wc_r, bc_r, ho_r, lg_r):
    bv = beta_r[...]
    hout = z0_r[...] * bv[0:1, 0:1] + z1_r[...] * bv[0:1, 1:2]
    ho_r[...] = hout
    lg_r[...] = jnp.dot(hout, wc_r[...], precision=HI,
                        preferred_element_type=jnp.float32) + bc_r[...]


def _stage5(z0, z1, betav, wc, bc):
    return pl.pallas_call(
        _t5_body,
        grid=(GRID,),
        in_specs=[_row_spec(OUT * HEADS), _row_spec(OUT * HEADS),
                  _full_spec(1, 8), _full_spec(OUT * HEADS, NC),
                  _full_spec(1, NC)],
        out_specs=[_row_spec(OUT * HEADS), _row_spec(NC)],
        out_shape=[jax.ShapeDtypeStruct((N, OUT * HEADS), jnp.float32),
                   jax.ShapeDtypeStruct((N, NC), jnp.float32)],
    )(z0, z1, betav, wc, bc)


def _head_proj(a):
    return (a[:, :, None] * jnp.eye(HEADS, dtype=jnp.float32)[:, None, :]
            ).reshape(HID, HEADS)


def _gvec_from(gm):
    g = gm[0] + gm[1]
    g = jnp.where(g < 0.0, 0.2 * g, g)
    return jnp.concatenate([g, jnp.zeros((1,), jnp.float32),
                            jnp.full((7,), 1e30, jnp.float32)])


@jax.jit
def kernel(feat0, params, adj, type_mask):
    del type_mask
    srcs = adj[0]
    dsts = adj[1]
    expand = jnp.repeat(jnp.eye(HEADS, dtype=jnp.float32), DH, axis=1)
    z144 = jnp.zeros((NP, EXTW), jnp.float32)
    z128 = jnp.zeros((NP, HALF), jnp.float32)
    pad8 = jnp.zeros((N, 8), jnp.float32)

    as1 = _head_proj(params['gat1_as'])
    ad1 = _head_proj(params['gat1_ad'])
    as2 = _head_proj(params['gat2_as'])
    ad2 = _head_proj(params['gat2_ad'])

    fs, hd1, es1, ed1, gm1 = _stage1(
        feat0, params['W_fc'], params['b_fc'][None, :],
        params['gat1_W'], as1, ad1)

    hdA1 = jnp.concatenate([hd1[:, :HALF], es1, pad8], axis=1)
    hdB1 = jnp.concatenate([hd1[:, HALF:], es1, pad8], axis=1)
    edt1 = jnp.concatenate([ed1, pad8], axis=1)
    acc1A, acc1B = _gat_edge(hdA1, hdB1, edt1, srcs, dsts,
                             _gvec_from(gm1), z144)
    acc1A, acc1B = acc1A[:N], acc1B[:N]

    hd2, es2, ed2, gm2 = _stage2(acc1A, acc1B, expand,
                                 params['gat2_W'], as2, ad2)

    hdA2 = jnp.concatenate([hd2[:, :HALF], es2, pad8], axis=1)
    hdB2 = jnp.concatenate([hd2[:, HALF:], es2, pad8], axis=1)
    edt2 = jnp.concatenate([ed2, pad8], axis=1)
    acc2A, acc2B = _gat_edge(hdA2, hdB2, edt2, srcs, dsts,
                             _gvec_from(gm2), z144)
    acc2A, acc2B = acc2A[:N], acc2B[:N]

    hA, hB, degc = _stage3(acc2A, acc2B, expand, fs)

    aggA, aggB = _agg_edge(hA, hB, srcs, dsts, z128)
    aggA, aggB = aggA[:N], aggB[:N]

    z0, z1, ss = _stage4(aggA, aggB, degc,
                         params['W_mp'][0], params['W_mp'][1],
                         params['b_mp'][0][None, :], params['b_mp'][1][None, :],
                         params['W_q'], params['b_q'][None, :],
                         jnp.concatenate(
                             [params['q'][:, None],
                              jnp.zeros((ATT, 7), jnp.float32)], axis=1))

    beta = jax.nn.softmax(ss[0:2, 0] / N)
    betav = jnp.concatenate([beta, jnp.zeros((6,), jnp.float32)])[None, :]

    h_out, logits = _stage5(z0, z1, betav, params['W_cls'],
                            params['b_cls'][None, :])
    return (logits, h_out)


# speedup vs baseline: 30.2669x; 1.7898x over previous
import jax
import jax.numpy as jnp
from jax import lax
from jax.experimental import pallas as pl
from jax.experimental.pallas import tpu as pltpu
from jax.experimental.pallas import tpu_sc as plsc

N = 10000
E = 320000
DF = 128
HID = 256
HEADS = 8
DH = 32
OUT = 128
NC = 8
M = 2
ATT = 128
SLICE = 4019

HALF = HID // 2
EXW = 16
NSUB = 16
EPT = E // NSUB
CHUNK = 80
NCHUNK = EPT // CHUNK
NP = 10240
RPT = NP // NSUB

BLK = 1000
GRID = N // BLK
HI = lax.Precision.HIGHEST

_GDN = lax.GatherDimensionNumbers(offset_dims=(), collapsed_slice_dims=(0,),
                                  start_index_map=(0,))


def _elu(x):
    return jnp.where(x > 0, x, jnp.exp(x) - 1.0)


def _bcast(v, sel):
    return lax.gather(v, jnp.full((16, 1), sel, jnp.int32),
                      dimension_numbers=_GDN, slice_sizes=(1,),
                      mode=lax.GatherScatterMode.PROMISE_IN_BOUNDS)


def _gat_edge_body(feat, esq, edq, srcs, dsts, gvec, zinit, zinit2,
                   accO, denO,
                   acc, acc2,
                   srcb0, dstb0, srcb1, dstb1, rows0, rows1, exb0, exb1,
                   esr0, esr1, edr0, edr1, dsc0, dsc1, gv,
                   sg0, sg1, si0, si1, ss0, ss1):
    c = lax.axis_index("c")
    s = lax.axis_index("s")
    pltpu.sync_copy(gvec.at[c], gv)
    pltpu.sync_copy(zinit.at[pl.ds(s * RPT, RPT)], acc.at[pl.ds(s * RPT, RPT)])
    pltpu.sync_copy(zinit2.at[pl.ds(s * RPT, RPT)], acc2.at[pl.ds(s * RPT, RPT)])
    plsc.subcore_barrier()

    g = gv[...]
    iot = lax.iota(jnp.int32, 16)
    qsel = iot // 4
    lane4 = iot % 4
    onescol = jnp.where(iot == 4, 1.0, 0.0).astype(jnp.float32)

    def initex(i, carry):
        i16 = jnp.full((16,), i, jnp.int32)
        plsc.store_scatter(exb0, [i16, iot], onescol)
        plsc.store_scatter(exb1, [i16, iot], onescol)
        return carry

    lax.fori_loop(0, CHUNK, initex, 0)

    def idx_start(k, sb, db, sem):
        base = s * EPT + k * CHUNK
        pltpu.async_copy(srcs.at[pl.ds(base, CHUNK)], sb, sem)
        pltpu.async_copy(dsts.at[pl.ds(base, CHUNK)], db, sem)

    def idx_wait(sb, db, sem):
        pltpu.make_async_copy(srcs.at[pl.ds(0, CHUNK)], sb, sem).wait()
        pltpu.make_async_copy(dsts.at[pl.ds(0, CHUNK)], db, sem).wait()

    def compute(sb, db, rw, exb, esr, edr, dsc):
        for t in range(CHUNK // 16):
            dsc[pl.ds(16 * t, 16)] = db[pl.ds(16 * t, 16)]

        def quad(q, carry):
            q16 = jnp.full((16,), 4 * q, jnp.int32) + qsel
            esv = plsc.load_gather(esr, [q16, lane4])
            edv = plsc.load_gather(edr, [q16, lane4])
            e = esv + edv
            e = jnp.where(e < 0.0, 0.2 * e, e)
            ex = jnp.exp(e - g)
            plsc.store_scatter(exb, [q16, lane4], ex)
            for r in range(4):
                ri = jnp.full((16,), 4 * q + r, jnp.int32)
                for j in range(HALF // 16):
                    bc = _bcast(ex, 4 * r + j // 2)
                    fv = plsc.load_gather(rw, [ri, j * 16 + iot])
                    plsc.store_scatter(rw, [ri, j * 16 + iot], fv * bc)
            return carry

        lax.fori_loop(0, CHUNK // 4, quad, 0)

    bufs = ((srcb0, dstb0, rows0, exb0, esr0, edr0, dsc0, sg0, si0, ss0),
            (srcb1, dstb1, rows1, exb1, esr1, edr1, dsc1, sg1, si1, ss1))

    def gat_start(sb, db, rw, esr, edr, sem):
        pltpu.async_copy(feat.at[c].at[sb], rw, sem)
        pltpu.async_copy(esq.at[c].at[sb], esr, sem)
        pltpu.async_copy(edq.at[c].at[db], edr, sem)

    def gat_wait(sb, db, rw, esr, edr, sem):
        pltpu.make_async_copy(feat.at[c].at[sb], rw, sem).wait()
        pltpu.make_async_copy(esq.at[c].at[sb], esr, sem).wait()
        pltpu.make_async_copy(edq.at[c].at[db], edr, sem).wait()

    pltpu.sync_copy(srcs.at[pl.ds(s * EPT, CHUNK)], srcb0)
    pltpu.sync_copy(dsts.at[pl.ds(s * EPT, CHUNK)], dstb0)
    gat_start(srcb0, dstb0, rows0, esr0, edr0, sg0)
    idx_start(1, srcb1, dstb1, si1)

    def pair(p, carry):
        for b in range(2):
            k = 2 * p + b
            sb, db, rw, exb, esr, edr, dsc, sg, si, ss = bufs[b]
            nsb, ndb, nrw, nexb, nesr, nedr, ndsc, nsg, nsi, nss = bufs[1 - b]
            gat_wait(sb, db, rw, esr, edr, sg)

            @pl.when(k + 1 < NCHUNK)
            def _():
                idx_wait(nsb, ndb, nsi)

                @pl.when(k >= 1)
                def _():
                    pltpu.make_async_copy(nrw, acc.at[ndsc], nss).wait()
                    pltpu.make_async_copy(nexb, acc2.at[ndsc], nss).wait()

                gat_start(nsb, ndb, nrw, nesr, nedr, nsg)

            compute(sb, db, rw, exb, esr, edr, dsc)
            pltpu.async_copy(rw, acc.at[dsc], ss, add=True)
            pltpu.async_copy(exb, acc2.at[dsc], ss, add=True)

            @pl.when(k + 2 < NCHUNK)
            def _():
                idx_start(k + 2, sb, db, si)
        return carry

    lax.fori_loop(0, NCHUNK // 2, pair, 0)
    pltpu.make_async_copy(rows0, acc.at[dsc0], ss0).wait()
    pltpu.make_async_copy(exb0, acc2.at[dsc0], ss0).wait()
    pltpu.make_async_copy(rows1, acc.at[dsc1], ss1).wait()
    pltpu.make_async_copy(exb1, acc2.at[dsc1], ss1).wait()
    plsc.subcore_barrier()

    pltpu.sync_copy(acc.at[pl.ds(s * RPT, RPT)],
                    accO.at[c].at[pl.ds(s * RPT, RPT)])
    pltpu.sync_copy(acc2.at[pl.ds(s * RPT, RPT)],
                    denO.at[c].at[pl.ds(s * RPT, RPT)])


_gat_edge = pl.kernel(
    _gat_edge_body,
    out_type=(jax.ShapeDtypeStruct((2, NP, HALF), jnp.float32),
              jax.ShapeDtypeStruct((2, NP, EXW), jnp.float32)),
    mesh=plsc.VectorSubcoreMesh(core_axis_name="c", subcore_axis_name="s"),
    compiler_params=pltpu.CompilerParams(needs_layout_passes=False,
                                         use_tc_tiling_on_sc=False),
    scratch_types=[
        pltpu.VMEM_SHARED((NP, HALF), jnp.float32),
        pltpu.VMEM_SHARED((NP, EXW), jnp.float32),
        pltpu.VMEM((CHUNK,), jnp.int32),
        pltpu.VMEM((CHUNK,), jnp.int32),
        pltpu.VMEM((CHUNK,), jnp.int32),
        pltpu.VMEM((CHUNK,), jnp.int32),
        pltpu.VMEM((CHUNK, HALF), jnp.float32),
        pltpu.VMEM((CHUNK, HALF), jnp.float32),
        pltpu.VMEM((CHUNK, EXW), jnp.float32),
        pltpu.VMEM((CHUNK, EXW), jnp.float32),
        pltpu.VMEM((CHUNK, 8), jnp.float32),
        pltpu.VMEM((CHUNK, 8), jnp.float32),
        pltpu.VMEM((CHUNK, 8), jnp.float32),
        pltpu.VMEM((CHUNK, 8), jnp.float32),
        pltpu.VMEM((CHUNK,), jnp.int32),
        pltpu.VMEM((CHUNK,), jnp.int32),
        pltpu.VMEM((16,), jnp.float32),
        pltpu.SemaphoreType.DMA,
        pltpu.SemaphoreType.DMA,
        pltpu.SemaphoreType.DMA,
        pltpu.SemaphoreType.DMA,
        pltpu.SemaphoreType.DMA,
        pltpu.SemaphoreType.DMA,
    ],
)


def _agg_edge_body(feat, srcs, dsts, zinit,
                   aggO,
                   acc, srcb0, dstb0, srcb1, dstb1, rows0, rows1, dsc0, dsc1,
                   sg0, sg1, si0, si1, ss0, ss1):
    c = lax.axis_index("c")
    s = lax.axis_index("s")
    pltpu.sync_copy(zinit.at[pl.ds(s * RPT, RPT)], acc.at[pl.ds(s * RPT, RPT)])
    plsc.subcore_barrier()

    def idx_start(k, sb, db, sem):
        base = s * EPT + k * CHUNK
        pltpu.async_copy(srcs.at[pl.ds(base, CHUNK)], sb, sem)
        pltpu.async_copy(dsts.at[pl.ds(base, CHUNK)], db, sem)

    bufs = ((srcb0, dstb0, rows0, dsc0, sg0, si0, ss0),
            (srcb1, dstb1, rows1, dsc1, sg1, si1, ss1))

    pltpu.sync_copy(srcs.at[pl.ds(s * EPT, CHUNK)], srcb0)
    pltpu.sync_copy(dsts.at[pl.ds(s * EPT, CHUNK)], dstb0)
    pltpu.async_copy(feat.at[c].at[srcb0], rows0, sg0)
    idx_start(1, srcb1, dstb1, si1)

    def pair(p, carry):
        for b in range(2):
            k = 2 * p + b
            sb, db, rw, dsc, sg, si, ss = bufs[b]
            nsb, ndb, nrw, ndsc, nsg, nsi, nss = bufs[1 - b]
            pltpu.make_async_copy(feat.at[c].at[sb], rw, sg).wait()

            @pl.when(k + 1 < NCHUNK)
            def _():
                pltpu.make_async_copy(srcs.at[pl.ds(0, CHUNK)], nsb, nsi).wait()
                pltpu.make_async_copy(dsts.at[pl.ds(0, CHUNK)], ndb, nsi).wait()

                @pl.when(k >= 1)
                def _():
                    pltpu.make_async_copy(nrw, acc.at[ndsc], nss).wait()

                pltpu.async_copy(feat.at[c].at[nsb], nrw, nsg)

            for t in range(CHUNK // 16):
                dsc[pl.ds(16 * t, 16)] = db[pl.ds(16 * t, 16)]
            pltpu.async_copy(rw, acc.at[dsc], ss, add=True)

            @pl.when(k + 2 < NCHUNK)
            def _():
                idx_start(k + 2, sb, db, si)
        return carry

    lax.fori_loop(0, NCHUNK // 2, pair, 0)
    pltpu.make_async_copy(rows0, acc.at[dsc0], ss0).wait()
    pltpu.make_async_copy(rows1, acc.at[dsc1], ss1).wait()
    plsc.subcore_barrier()

    pltpu.sync_copy(acc.at[pl.ds(s * RPT, RPT)],
                    aggO.at[c].at[pl.ds(s * RPT, RPT)])


_agg_edge = pl.kernel(
    _agg_edge_body,
    out_type=jax.ShapeDtypeStruct((2, NP, HALF), jnp.float32),
    mesh=plsc.VectorSubcoreMesh(core_axis_name="c", subcore_axis_name="s"),
    compiler_params=pltpu.CompilerParams(needs_layout_passes=False,
                                         use_tc_tiling_on_sc=False),
    scratch_types=[
        pltpu.VMEM_SHARED((NP, HALF), jnp.float32),
        pltpu.VMEM((CHUNK,), jnp.int32),
        pltpu.VMEM((CHUNK,), jnp.int32),
        pltpu.VMEM((CHUNK,), jnp.int32),
        pltpu.VMEM((CHUNK,), jnp.int32),
        pltpu.VMEM((CHUNK, HALF), jnp.float32),
        pltpu.VMEM((CHUNK, HALF), jnp.float32),
        pltpu.VMEM((CHUNK,), jnp.int32),
        pltpu.VMEM((CHUNK,), jnp.int32),
        pltpu.SemaphoreType.DMA,
        pltpu.SemaphoreType.DMA,
        pltpu.SemaphoreType.DMA,
        pltpu.SemaphoreType.DMA,
        pltpu.SemaphoreType.DMA,
        pltpu.SemaphoreType.DMA,
    ],
)


def _row_spec(w):
    return pl.BlockSpec((BLK, w), lambda i: (i, 0))


def _full_spec(h, w):
    return pl.BlockSpec((h, w), lambda i: (0, 0))


def _t1_body(x_r, wfc_r, bfc_r, w1_r, as_r, ad_r,
             fs_r, hd_r, es_r, ed_r, gm_r):
    i = pl.program_id(0)
    fs = jnp.dot(x_r[...], wfc_r[...], precision=HI,
                 preferred_element_type=jnp.float32) + bfc_r[...]
    hd = jnp.dot(fs, w1_r[...], precision=HI,
                 preferred_element_type=jnp.float32)
    es = jnp.dot(hd, as_r[...], precision=HI,
                 preferred_element_type=jnp.float32)
    ed = jnp.dot(hd, ad_r[...], precision=HI,
                 preferred_element_type=jnp.float32)
    fs_r[...] = fs
    hd_r[...] = hd
    es_r[...] = es
    ed_r[...] = ed
    new = jnp.concatenate([jnp.max(es, axis=0, keepdims=True),
                           jnp.max(ed, axis=0, keepdims=True),
                           jnp.full((6, HEADS), -1e30, jnp.float32)], axis=0)

    @pl.when(i == 0)
    def _():
        gm_r[...] = new

    @pl.when(i > 0)
    def _():
        gm_r[...] = jnp.maximum(gm_r[...], new)


def _stage1(feat0, wfc, bfc, w1, a_s, a_d):
    return pl.pallas_call(
        _t1_body,
        grid=(GRID,),
        in_specs=[_row_spec(DF), _full_spec(DF, HID), _full_spec(1, HID),
                  _full_spec(HID, HID), _full_spec(HID, HEADS),
                  _full_spec(HID, HEADS)],
        out_specs=[_row_spec(HID), _row_spec(HID), _row_spec(HEADS),
                   _row_spec(HEADS), _full_spec(8, HEADS)],
        out_shape=[jax.ShapeDtypeStruct((N, HID), jnp.float32),
                   jax.ShapeDtypeStruct((N, HID), jnp.float32),
                   jax.ShapeDtypeStruct((N, HEADS), jnp.float32),
                   jax.ShapeDtypeStruct((N, HEADS), jnp.float32),
                   jax.ShapeDtypeStruct((8, HEADS), jnp.float32)],
    )(feat0, wfc, bfc, w1, a_s, a_d)


def _gat_out(aA, aB, dA, dB, expand):
    den = jnp.concatenate([dA[:, 0:4], dB[:, 0:4]], axis=1) + 1e-9
    denx = jnp.dot(den, expand, precision=HI,
                   preferred_element_type=jnp.float32)
    num = jnp.concatenate([aA, aB], axis=1)
    return num / denx


def _t2_body(aA_r, aB_r, dA_r, dB_r, exp_r, w2_r, as_r, ad_r,
             hd_r, es_r, ed_r, gm_r):
    i = pl.program_id(0)
    h1 = _elu(_gat_out(aA_r[...], aB_r[...], dA_r[...], dB_r[...], exp_r[...]))
    hd = jnp.dot(h1, w2_r[...], precision=HI,
                 preferred_element_type=jnp.float32)
    es = jnp.dot(hd, as_r[...], precision=HI,
                 preferred_element_type=jnp.float32)
    ed = jnp.dot(hd, ad_r[...], precision=HI,
                 preferred_element_type=jnp.float32)
    hd_r[...] = hd
    es_r[...] = es
    ed_r[...] = ed
    new = jnp.concatenate([jnp.max(es, axis=0, keepdims=True),
                           jnp.max(ed, axis=0, keepdims=True),
                           jnp.full((6, HEADS), -1e30, jnp.float32)], axis=0)

    @pl.when(i == 0)
    def _():
        gm_r[...] = new

    @pl.when(i > 0)
    def _():
        gm_r[...] = jnp.maximum(gm_r[...], new)


def _stage2(accA, accB, denA, denB, expand, w2, a_s, a_d):
    return pl.pallas_call(
        _t2_body,
        grid=(GRID,),
        in_specs=[_row_spec(HALF), _row_spec(HALF), _row_spec(EXW),
                  _row_spec(EXW), _full_spec(HEADS, HID),
                  _full_spec(HID, HID), _full_spec(HID, HEADS),
                  _full_spec(HID, HEADS)],
        out_specs=[_row_spec(HID), _row_spec(HEADS), _row_spec(HEADS),
                   _full_spec(8, HEADS)],
        out_shape=[jax.ShapeDtypeStruct((N, HID), jnp.float32),
                   jax.ShapeDtypeStruct((N, HEADS), jnp.float32),
                   jax.ShapeDtypeStruct((N, HEADS), jnp.float32),
                   jax.ShapeDtypeStruct((8, HEADS), jnp.float32)],
    )(accA, accB, denA, denB, expand, w2, a_s, a_d)


def _t3_body(aA_r, aB_r, dA_r, dB_r, exp_r, fs_r, hA_r, hB_r, deg_r):
    i = pl.program_id(0)
    h2 = _gat_out(aA_r[...], aB_r[...], dA_r[...], dB_r[...], exp_r[...])
    ridx = lax.broadcasted_iota(jnp.int32, (BLK, 1), 0) + i * BLK
    h2 = jnp.where(ridx < SLICE, fs_r[...], h2)
    hA_r[...] = h2[:, :HALF]
    hB_r[...] = h2[:, HALF:]
    deg_r[...] = dA_r[...][:, 4:12]


def _stage3(accA, accB, denA, denB, expand, fs):
    return pl.pallas_call(
        _t3_body,
        grid=(GRID,),
        in_specs=[_row_spec(HALF), _row_spec(HALF), _row_spec(EXW),
                  _row_spec(EXW), _full_spec(HEADS, HID), _row_spec(HID)],
        out_specs=[_row_spec(HALF), _row_spec(HALF), _row_spec(8)],
        out_shape=[jax.ShapeDtypeStruct((N, HALF), jnp.float32),
                   jax.ShapeDtypeStruct((N, HALF), jnp.float32),
                   jax.ShapeDtypeStruct((N, 8), jnp.float32)],
    )(accA, accB, denA, denB, expand, fs)


def _t4_body(hA_r, hB_r, deg_r, w0_r, w1_r, b0_r, b1_r, wq_r, bq_r, q_r,
             z0_r, z1_r, ss_r):
    i = pl.program_id(0)
    deg = jnp.maximum(deg_r[...][:, 0:1], 1.0)
    aggv = jnp.concatenate([hA_r[...], hB_r[...]], axis=1) / deg
    z0 = _elu(jnp.dot(aggv, w0_r[...], precision=HI,
                      preferred_element_type=jnp.float32) + b0_r[...])
    z1 = _elu(jnp.dot(aggv, w1_r[...], precision=HI,
                      preferred_element_type=jnp.float32) + b1_r[...])
    z0_r[...] = z0
    z1_r[...] = z1
    t0 = jnp.tanh(jnp.dot(z0, wq_r[...], precision=HI,
                          preferred_element_type=jnp.float32) + bq_r[...])
    t1 = jnp.tanh(jnp.dot(z1, wq_r[...], precision=HI,
                          preferred_element_type=jnp.float32) + bq_r[...])
    s0 = jnp.dot(t0, q_r[...], precision=HI,
                 preferred_element_type=jnp.float32)
    s1 = jnp.dot(t1, q_r[...], precision=HI,
                 preferred_element_type=jnp.float32)
    new = jnp.concatenate([jnp.sum(s0, axis=0, keepdims=True),
                           jnp.sum(s1, axis=0, keepdims=True),
                           jnp.zeros((6, 8), jnp.float32)], axis=0)

    @pl.when(i == 0)
    def _():
        ss_r[...] = new

    @pl.when(i > 0)
    def _():
        ss_r[...] = ss_r[...] + new


def _stage4(hA, hB, degc, w0, w1, b0, b1, wq, bq, qv):
    return pl.pallas_call(
        _t4_body,
        grid=(GRID,),
        in_specs=[_row_spec(HALF), _row_spec(HALF), _row_spec(8),
                  _full_spec(HID, OUT * HEADS), _full_spec(HID, OUT * HEADS),
                  _full_spec(1, OUT * HEADS), _full_spec(1, OUT * HEADS),
                  _full_spec(OUT * HEADS, ATT), _full_spec(1, ATT),
                  _full_spec(ATT, 8)],
        out_specs=[_row_spec(OUT * HEADS), _row_spec(OUT * HEADS),
                   _full_spec(8, 8)],
        out_shape=[jax.ShapeDtypeStruct((N, OUT * HEADS), jnp.float32),
                   jax.ShapeDtypeStruct((N, OUT * HEADS), jnp.float32),
                   jax.ShapeDtypeStruct((8, 8), jnp.float32)],
    )(hA, hB, degc, w0, w1, b0, b1, wq, bq, qv)


def _t5_body(z0_r, z1_r, beta_r, wc_r, bc_r, ho_r, lg_r):
    bv = beta_r[...]
    hout = z0_r[...] * bv[0:1, 0:1] + z1_r[...] * bv[0:1, 1:2]
    ho_r[...] = hout
    lg_r[...] = jnp.dot(hout, wc_r[...], precision=HI,
                        preferred_element_type=jnp.float32) + bc_r[...]


def _stage5(z0, z1, betav, wc, bc):
    return pl.pallas_call(
        _t5_body,
        grid=(GRID,),
        in_specs=[_row_spec(OUT * HEADS), _row_spec(OUT * HEADS),
                  _full_spec(1, 8), _full_spec(OUT * HEADS, NC),
                  _full_spec(1, NC)],
        out_specs=[_row_spec(OUT * HEADS), _row_spec(NC)],
        out_shape=[jax.ShapeDtypeStruct((N, OUT * HEADS), jnp.float32),
                   jax.ShapeDtypeStruct((N, NC), jnp.float32)],
    )(z0, z1, betav, wc, bc)


def _head_proj(a):
    return (a[:, :, None] * jnp.eye(HEADS, dtype=jnp.float32)[:, None, :]
            ).reshape(HID, HEADS)


def _gvec_from(gm):
    g = gm[0] + gm[1]
    g = jnp.where(g < 0.0, 0.2 * g, g)
    return jnp.stack([jnp.tile(g[0:4], 4), jnp.tile(g[4:8], 4)])


def _edge_tables(hd, es, ed):
    feats = jnp.stack([hd[:, :HALF], hd[:, HALF:]])
    pad4 = jnp.zeros((es.shape[0], 4), jnp.float32)
    esq = jnp.stack([jnp.concatenate([es[:, 0:4], pad4], axis=1),
                     jnp.concatenate([es[:, 4:8], pad4], axis=1)])
    edq = jnp.stack([jnp.concatenate([ed[:, 0:4], pad4], axis=1),
                     jnp.concatenate([ed[:, 4:8], pad4], axis=1)])
    return feats, esq, edq


@jax.jit
def kernel(feat0, params, adj, type_mask):
    del type_mask
    srcs = adj[0]
    dsts = adj[1]
    expand = jnp.repeat(jnp.eye(HEADS, dtype=jnp.float32), DH, axis=1)
    z128 = jnp.zeros((NP, HALF), jnp.float32)
    z16 = jnp.zeros((NP, EXW), jnp.float32)

    as1 = _head_proj(params['gat1_as'])
    ad1 = _head_proj(params['gat1_ad'])
    as2 = _head_proj(params['gat2_as'])
    ad2 = _head_proj(params['gat2_ad'])

    fs, hd1, es1, ed1, gm1 = _stage1(
        feat0, params['W_fc'], params['b_fc'][None, :],
        params['gat1_W'], as1, ad1)

    f1, esq1, edq1 = _edge_tables(hd1, es1, ed1)
    accO1, denO1 = _gat_edge(f1, esq1, edq1, srcs, dsts,
                             _gvec_from(gm1), z128, z16)

    hd2, es2, ed2, gm2 = _stage2(accO1[0, :N], accO1[1, :N],
                                 denO1[0, :N], denO1[1, :N],
                                 expand, params['gat2_W'], as2, ad2)

    f2, esq2, edq2 = _edge_tables(hd2, es2, ed2)
    accO2, denO2 = _gat_edge(f2, esq2, edq2, srcs, dsts,
                             _gvec_from(gm2), z128, z16)

    hA, hB, degc = _stage3(accO2[0, :N], accO2[1, :N],
                           denO2[0, :N], denO2[1, :N], expand, fs)

    aggO = _agg_edge(jnp.stack([hA, hB]), srcs, dsts, z128)

    z0, z1, ss = _stage4(aggO[0, :N], aggO[1, :N], degc,
                         params['W_mp'][0], params['W_mp'][1],
                         params['b_mp'][0][None, :], params['b_mp'][1][None, :],
                         params['W_q'], params['b_q'][None, :],
                         jnp.concatenate(
                             [params['q'][:, None],
                              jnp.zeros((ATT, 7), jnp.float32)], axis=1))

    beta = jax.nn.softmax(ss[0:2, 0] / N)
    betav = jnp.concatenate([beta, jnp.zeros((6,), jnp.float32)])[None, :]

    h_out, logits = _stage5(z0, z1, betav, params['W_cls'],
                            params['b_cls'][None, :])
    return (logits, h_out)


# speedup vs baseline: 30.4993x; 1.0077x over previous
import jax
import jax.numpy as jnp
from jax import lax
from jax.experimental import pallas as pl
from jax.experimental.pallas import tpu as pltpu
from jax.experimental.pallas import tpu_sc as plsc

N = 10000
E = 320000
DF = 128
HID = 256
HEADS = 8
DH = 32
OUT = 128
NC = 8
M = 2
ATT = 128
SLICE = 4019

HALF = HID // 2
EXW = 16
NSUB = 16
EPT = E // NSUB
CHUNK = 80
NCHUNK = EPT // CHUNK
NP = 10240
RPT = NP // NSUB

BLK = 1000
GRID = N // BLK
HI = lax.Precision.HIGHEST

_GDN = lax.GatherDimensionNumbers(offset_dims=(), collapsed_slice_dims=(0,),
                                  start_index_map=(0,))


def _elu(x):
    return jnp.where(x > 0, x, jnp.exp(x) - 1.0)


def _bcast(v, sel):
    return lax.gather(v, jnp.full((16, 1), sel, jnp.int32),
                      dimension_numbers=_GDN, slice_sizes=(1,),
                      mode=lax.GatherScatterMode.PROMISE_IN_BOUNDS)


def _gat_edge_body(feat, esq, edq, srcs, dsts, gvec, zinit, zinit2,
                   accO, denO,
                   acc, acc2,
                   srcb0, dstb0, srcb1, dstb1, rows0, rows1, exb0, exb1,
                   esr0, esr1, edr0, edr1, dsc0, dsc1, gv,
                   sg0, sg1, si0, si1, ss0, ss1):
    c = lax.axis_index("c")
    s = lax.axis_index("s")
    pltpu.sync_copy(gvec.at[c], gv)
    pltpu.sync_copy(zinit.at[pl.ds(s * RPT, RPT)], acc.at[pl.ds(s * RPT, RPT)])
    pltpu.sync_copy(zinit2.at[pl.ds(s * RPT, RPT)], acc2.at[pl.ds(s * RPT, RPT)])
    plsc.subcore_barrier()

    g = gv[...]
    iot = lax.iota(jnp.int32, 16)
    qsel = iot // 4
    lane4 = iot % 4
    onescol = jnp.where(iot == 4, 1.0, 0.0).astype(jnp.float32)

    def initex(i, carry):
        i16 = jnp.full((16,), i, jnp.int32)
        plsc.store_scatter(exb0, [i16, iot], onescol)
        plsc.store_scatter(exb1, [i16, iot], onescol)
        return carry

    lax.fori_loop(0, CHUNK, initex, 0)

    def idx_start(k, sb, db, sem):
        base = s * EPT + k * CHUNK
        pltpu.async_copy(srcs.at[pl.ds(base, CHUNK)], sb, sem)
        pltpu.async_copy(dsts.at[pl.ds(base, CHUNK)], db, sem)

    def idx_wait(sb, db, sem):
        pltpu.make_async_copy(srcs.at[pl.ds(0, CHUNK)], sb, sem).wait()
        pltpu.make_async_copy(dsts.at[pl.ds(0, CHUNK)], db, sem).wait()

    def compute(sb, db, rw, exb, esr, edr, dsc):
        for t in range(CHUNK // 16):
            dsc[pl.ds(16 * t, 16)] = db[pl.ds(16 * t, 16)]

        def quad(q):
            q16 = jnp.full((16,), 4 * q, jnp.int32) + qsel
            esv = plsc.load_gather(esr, [q16, lane4])
            edv = plsc.load_gather(edr, [q16, lane4])
            e = esv + edv
            e = jnp.where(e < 0.0, 0.2 * e, e)
            ex = jnp.exp(e - g)
            plsc.store_scatter(exb, [q16, lane4], ex)
            bcs = [_bcast(ex, i) for i in range(16)]
            for r in range(4):
                ri = jnp.full((16,), 4 * q + r, jnp.int32)
                for j in range(HALF // 16):
                    fv = plsc.load_gather(rw, [ri, j * 16 + iot])
                    plsc.store_scatter(rw, [ri, j * 16 + iot],
                                       fv * bcs[4 * r + j // 2])

        def quad2(q, carry):
            quad(2 * q)
            quad(2 * q + 1)
            return carry

        lax.fori_loop(0, CHUNK // 8, quad2, 0)

    bufs = ((srcb0, dstb0, rows0, exb0, esr0, edr0, dsc0, sg0, si0, ss0),
            (srcb1, dstb1, rows1, exb1, esr1, edr1, dsc1, sg1, si1, ss1))

    def gat_start(sb, db, rw, esr, edr, sem):
        pltpu.async_copy(feat.at[c].at[sb], rw, sem)
        pltpu.async_copy(esq.at[c].at[sb], esr, sem)
        pltpu.async_copy(edq.at[c].at[db], edr, sem)

    def gat_wait(sb, db, rw, esr, edr, sem):
        pltpu.make_async_copy(feat.at[c].at[sb], rw, sem).wait()
        pltpu.make_async_copy(esq.at[c].at[sb], esr, sem).wait()
        pltpu.make_async_copy(edq.at[c].at[db], edr, sem).wait()

    pltpu.sync_copy(srcs.at[pl.ds(s * EPT, CHUNK)], srcb0)
    pltpu.sync_copy(dsts.at[pl.ds(s * EPT, CHUNK)], dstb0)
    gat_start(srcb0, dstb0, rows0, esr0, edr0, sg0)
    idx_start(1, srcb1, dstb1, si1)

    def pair(p, carry):
        for b in range(2):
            k = 2 * p + b
            sb, db, rw, exb, esr, edr, dsc, sg, si, ss = bufs[b]
            nsb, ndb, nrw, nexb, nesr, nedr, ndsc, nsg, nsi, nss = bufs[1 - b]
            gat_wait(sb, db, rw, esr, edr, sg)

            @pl.when(k + 1 < NCHUNK)
            def _():
                idx_wait(nsb, ndb, nsi)

                @pl.when(k >= 1)
                def _():
                    pltpu.make_async_copy(nrw, acc.at[ndsc], nss).wait()
                    pltpu.make_async_copy(nexb, acc2.at[ndsc], nss).wait()

                gat_start(nsb, ndb, nrw, nesr, nedr, nsg)

            compute(sb, db, rw, exb, esr, edr, dsc)
            pltpu.async_copy(rw, acc.at[dsc], ss, add=True)
            pltpu.async_copy(exb, acc2.at[dsc], ss, add=True)

            @pl.when(k + 2 < NCHUNK)
            def _():
                idx_start(k + 2, sb, db, si)
        return carry

    lax.fori_loop(0, NCHUNK // 2, pair, 0)
    pltpu.make_async_copy(rows0, acc.at[dsc0], ss0).wait()
    pltpu.make_async_copy(exb0, acc2.at[dsc0], ss0).wait()
    pltpu.make_async_copy(rows1, acc.at[dsc1], ss1).wait()
    pltpu.make_async_copy(exb1, acc2.at[dsc1], ss1).wait()
    plsc.subcore_barrier()

    pltpu.sync_copy(acc.at[pl.ds(s * RPT, RPT)],
                    accO.at[c].at[pl.ds(s * RPT, RPT)])
    pltpu.sync_copy(acc2.at[pl.ds(s * RPT, RPT)],
                    denO.at[c].at[pl.ds(s * RPT, RPT)])


_gat_edge = pl.kernel(
    _gat_edge_body,
    out_type=(jax.ShapeDtypeStruct((2, NP, HALF), jnp.float32),
              jax.ShapeDtypeStruct((2, NP, EXW), jnp.float32)),
    mesh=plsc.VectorSubcoreMesh(core_axis_name="c", subcore_axis_name="s"),
    compiler_params=pltpu.CompilerParams(needs_layout_passes=False,
                                         use_tc_tiling_on_sc=False),
    scratch_types=[
        pltpu.VMEM_SHARED((NP, HALF), jnp.float32),
        pltpu.VMEM_SHARED((NP, EXW), jnp.float32),
        pltpu.VMEM((CHUNK,), jnp.int32),
        pltpu.VMEM((CHUNK,), jnp.int32),
        pltpu.VMEM((CHUNK,), jnp.int32),
        pltpu.VMEM((CHUNK,), jnp.int32),
        pltpu.VMEM((CHUNK, HALF), jnp.float32),
        pltpu.VMEM((CHUNK, HALF), jnp.float32),
        pltpu.VMEM((CHUNK, EXW), jnp.float32),
        pltpu.VMEM((CHUNK, EXW), jnp.float32),
        pltpu.VMEM((CHUNK, 8), jnp.float32),
        pltpu.VMEM((CHUNK, 8), jnp.float32),
        pltpu.VMEM((CHUNK, 8), jnp.float32),
        pltpu.VMEM((CHUNK, 8), jnp.float32),
        pltpu.VMEM((CHUNK,), jnp.int32),
        pltpu.VMEM((CHUNK,), jnp.int32),
        pltpu.VMEM((16,), jnp.float32),
        pltpu.SemaphoreType.DMA,
        pltpu.SemaphoreType.DMA,
        pltpu.SemaphoreType.DMA,
        pltpu.SemaphoreType.DMA,
        pltpu.SemaphoreType.DMA,
        pltpu.SemaphoreType.DMA,
    ],
)


def _agg_edge_body(feat, srcs, dsts, zinit,
                   aggO,
                   acc, srcb0, dstb0, srcb1, dstb1, rows0, rows1, dsc0, dsc1,
                   sg0, sg1, si0, si1, ss0, ss1):
    c = lax.axis_index("c")
    s = lax.axis_index("s")
    pltpu.sync_copy(zinit.at[pl.ds(s * RPT, RPT)], acc.at[pl.ds(s * RPT, RPT)])
    plsc.subcore_barrier()

    def idx_start(k, sb, db, sem):
        base = s * EPT + k * CHUNK
        pltpu.async_copy(srcs.at[pl.ds(base, CHUNK)], sb, sem)
        pltpu.async_copy(dsts.at[pl.ds(base, CHUNK)], db, sem)

    bufs = ((srcb0, dstb0, rows0, dsc0, sg0, si0, ss0),
            (srcb1, dstb1, rows1, dsc1, sg1, si1, ss1))

    pltpu.sync_copy(srcs.at[pl.ds(s * EPT, CHUNK)], srcb0)
    pltpu.sync_copy(dsts.at[pl.ds(s * EPT, CHUNK)], dstb0)
    pltpu.async_copy(feat.at[c].at[srcb0], rows0, sg0)
    idx_start(1, srcb1, dstb1, si1)

    def pair(p, carry):
        for b in range(2):
            k = 2 * p + b
            sb, db, rw, dsc, sg, si, ss = bufs[b]
            nsb, ndb, nrw, ndsc, nsg, nsi, nss = bufs[1 - b]
            pltpu.make_async_copy(feat.at[c].at[sb], rw, sg).wait()

            @pl.when(k + 1 < NCHUNK)
            def _():
                pltpu.make_async_copy(srcs.at[pl.ds(0, CHUNK)], nsb, nsi).wait()
                pltpu.make_async_copy(dsts.at[pl.ds(0, CHUNK)], ndb, nsi).wait()

                @pl.when(k >= 1)
                def _():
                    pltpu.make_async_copy(nrw, acc.at[ndsc], nss).wait()

                pltpu.async_copy(feat.at[c].at[nsb], nrw, nsg)

            for t in range(CHUNK // 16):
                dsc[pl.ds(16 * t, 16)] = db[pl.ds(16 * t, 16)]
            pltpu.async_copy(rw, acc.at[dsc], ss, add=True)

            @pl.when(k + 2 < NCHUNK)
            def _():
                idx_start(k + 2, sb, db, si)
        return carry

    lax.fori_loop(0, NCHUNK // 2, pair, 0)
    pltpu.make_async_copy(rows0, acc.at[dsc0], ss0).wait()
    pltpu.make_async_copy(rows1, acc.at[dsc1], ss1).wait()
    plsc.subcore_barrier()

    pltpu.sync_copy(acc.at[pl.ds(s * RPT, RPT)],
                    aggO.at[c].at[pl.ds(s * RPT, RPT)])


_agg_edge = pl.kernel(
    _agg_edge_body,
    out_type=jax.ShapeDtypeStruct((2, NP, HALF), jnp.float32),
    mesh=plsc.VectorSubcoreMesh(core_axis_name="c", subcore_axis_name="s"),
    compiler_params=pltpu.CompilerParams(needs_layout_passes=False,
                                         use_tc_tiling_on_sc=False),
    scratch_types=[
        pltpu.VMEM_SHARED((NP, HALF), jnp.float32),
        pltpu.VMEM((CHUNK,), jnp.int32),
        pltpu.VMEM((CHUNK,), jnp.int32),
        pltpu.VMEM((CHUNK,), jnp.int32),
        pltpu.VMEM((CHUNK,), jnp.int32),
        pltpu.VMEM((CHUNK, HALF), jnp.float32),
        pltpu.VMEM((CHUNK, HALF), jnp.float32),
        pltpu.VMEM((CHUNK,), jnp.int32),
        pltpu.VMEM((CHUNK,), jnp.int32),
        pltpu.SemaphoreType.DMA,
        pltpu.SemaphoreType.DMA,
        pltpu.SemaphoreType.DMA,
        pltpu.SemaphoreType.DMA,
        pltpu.SemaphoreType.DMA,
        pltpu.SemaphoreType.DMA,
    ],
)


def _row_spec(w):
    return pl.BlockSpec((BLK, w), lambda i: (i, 0))


def _full_spec(h, w):
    return pl.BlockSpec((h, w), lambda i: (0, 0))


def _t1_body(x_r, wfc_r, bfc_r, w1_r, as_r, ad_r,
             fs_r, hd_r, es_r, ed_r, gm_r):
    i = pl.program_id(0)
    fs = jnp.dot(x_r[...], wfc_r[...], precision=HI,
                 preferred_element_type=jnp.float32) + bfc_r[...]
    hd = jnp.dot(fs, w1_r[...], precision=HI,
                 preferred_element_type=jnp.float32)
    es = jnp.dot(hd, as_r[...], precision=HI,
                 preferred_element_type=jnp.float32)
    ed = jnp.dot(hd, ad_r[...], precision=HI,
                 preferred_element_type=jnp.float32)
    fs_r[...] = fs
    hd_r[...] = hd
    es_r[...] = es
    ed_r[...] = ed
    new = jnp.concatenate([jnp.max(es, axis=0, keepdims=True),
                           jnp.max(ed, axis=0, keepdims=True),
                           jnp.full((6, HEADS), -1e30, jnp.float32)], axis=0)

    @pl.when(i == 0)
    def _():
        gm_r[...] = new

    @pl.when(i > 0)
    def _():
        gm_r[...] = jnp.maximum(gm_r[...], new)


def _stage1(feat0, wfc, bfc, w1, a_s, a_d):
    return pl.pallas_call(
        _t1_body,
        grid=(GRID,),
        in_specs=[_row_spec(DF), _full_spec(DF, HID), _full_spec(1, HID),
                  _full_spec(HID, HID), _full_spec(HID, HEADS),
                  _full_spec(HID, HEADS)],
        out_specs=[_row_spec(HID), _row_spec(HID), _row_spec(HEADS),
                   _row_spec(HEADS), _full_spec(8, HEADS)],
        out_shape=[jax.ShapeDtypeStruct((N, HID), jnp.float32),
                   jax.ShapeDtypeStruct((N, HID), jnp.float32),
                   jax.ShapeDtypeStruct((N, HEADS), jnp.float32),
                   jax.ShapeDtypeStruct((N, HEADS), jnp.float32),
                   jax.ShapeDtypeStruct((8, HEADS), jnp.float32)],
    )(feat0, wfc, bfc, w1, a_s, a_d)


def _gat_out(aA, aB, dA, dB, expand):
    den = jnp.concatenate([dA[:, 0:4], dB[:, 0:4]], axis=1) + 1e-9
    denx = jnp.dot(den, expand, precision=HI,
                   preferred_element_type=jnp.float32)
    num = jnp.concatenate([aA, aB], axis=1)
    return num / denx


def _t2_body(aA_r, aB_r, dA_r, dB_r, exp_r, w2_r, as_r, ad_r,
             hd_r, es_r, ed_r, gm_r):
    i = pl.program_id(0)
    h1 = _elu(_gat_out(aA_r[...], aB_r[...], dA_r[...], dB_r[...], exp_r[...]))
    hd = jnp.dot(h1, w2_r[...], precision=HI,
                 preferred_element_type=jnp.float32)
    es = jnp.dot(hd, as_r[...], precision=HI,
                 preferred_element_type=jnp.float32)
    ed = jnp.dot(hd, ad_r[...], precision=HI,
                 preferred_element_type=jnp.float32)
    hd_r[...] = hd
    es_r[...] = es
    ed_r[...] = ed
    new = jnp.concatenate([jnp.max(es, axis=0, keepdims=True),
                           jnp.max(ed, axis=0, keepdims=True),
                           jnp.full((6, HEADS), -1e30, jnp.float32)], axis=0)

    @pl.when(i == 0)
    def _():
        gm_r[...] = new

    @pl.when(i > 0)
    def _():
        gm_r[...] = jnp.maximum(gm_r[...], new)


def _stage2(accA, accB, denA, denB, expand, w2, a_s, a_d):
    return pl.pallas_call(
        _t2_body,
        grid=(GRID,),
        in_specs=[_row_spec(HALF), _row_spec(HALF), _row_spec(EXW),
                  _row_spec(EXW), _full_spec(HEADS, HID),
                  _full_spec(HID, HID), _full_spec(HID, HEADS),
                  _full_spec(HID, HEADS)],
        out_specs=[_row_spec(HID), _row_spec(HEADS), _row_spec(HEADS),
                   _full_spec(8, HEADS)],
        out_shape=[jax.ShapeDtypeStruct((N, HID), jnp.float32),
                   jax.ShapeDtypeStruct((N, HEADS), jnp.float32),
                   jax.ShapeDtypeStruct((N, HEADS), jnp.float32),
                   jax.ShapeDtypeStruct((8, HEADS), jnp.float32)],
    )(accA, accB, denA, denB, expand, w2, a_s, a_d)


def _t3_body(aA_r, aB_r, dA_r, dB_r, exp_r, fs_r, hA_r, hB_r, deg_r):
    i = pl.program_id(0)
    h2 = _gat_out(aA_r[...], aB_r[...], dA_r[...], dB_r[...], exp_r[...])
    ridx = lax.broadcasted_iota(jnp.int32, (BLK, 1), 0) + i * BLK
    h2 = jnp.where(ridx < SLICE, fs_r[...], h2)
    hA_r[...] = h2[:, :HALF]
    hB_r[...] = h2[:, HALF:]
    deg_r[...] = dA_r[...][:, 4:12]


def _stage3(accA, accB, denA, denB, expand, fs):
    return pl.pallas_call(
        _t3_body,
        grid=(GRID,),
        in_specs=[_row_spec(HALF), _row_spec(HALF), _row_spec(EXW),
                  _row_spec(EXW), _full_spec(HEADS, HID), _row_spec(HID)],
        out_specs=[_row_spec(HALF), _row_spec(HALF), _row_spec(8)],
        out_shape=[jax.ShapeDtypeStruct((N, HALF), jnp.float32),
                   jax.ShapeDtypeStruct((N, HALF), jnp.float32),
                   jax.ShapeDtypeStruct((N, 8), jnp.float32)],
    )(accA, accB, denA, denB, expand, fs)


def _t4_body(hA_r, hB_r, deg_r, w0_r, w1_r, b0_r, b1_r, wq_r, bq_r, q_r,
             z0_r, z1_r, ss_r):
    i = pl.program_id(0)
    deg = jnp.maximum(deg_r[...][:, 0:1], 1.0)
    aggv = jnp.concatenate([hA_r[...], hB_r[...]], axis=1) / deg
    z0 = _elu(jnp.dot(aggv, w0_r[...], precision=HI,
                      preferred_element_type=jnp.float32) + b0_r[...])
    z1 = _elu(jnp.dot(aggv, w1_r[...], precision=HI,
                      preferred_element_type=jnp.float32) + b1_r[...])
    z0_r[...] = z0
    z1_r[...] = z1
    t0 = jnp.tanh(jnp.dot(z0, wq_r[...], precision=HI,
                          preferred_element_type=jnp.float32) + bq_r[...])
    t1 = jnp.tanh(jnp.dot(z1, wq_r[...], precision=HI,
                          preferred_element_type=jnp.float32) + bq_r[...])
    s0 = jnp.dot(t0, q_r[...], precision=HI,
                 preferred_element_type=jnp.float32)
    s1 = jnp.dot(t1, q_r[...], precision=HI,
                 preferred_element_type=jnp.float32)
    new = jnp.concatenate([jnp.sum(s0, axis=0, keepdims=True),
                           jnp.sum(s1, axis=0, keepdims=True),
                           jnp.zeros((6, 8), jnp.float32)], axis=0)

    @pl.when(i == 0)
    def _():
        ss_r[...] = new

    @pl.when(i > 0)
    def _():
        ss_r[...] = ss_r[...] + new


def _stage4(hA, hB, degc, w0, w1, b0, b1, wq, bq, qv):
    return pl.pallas_call(
        _t4_body,
        grid=(GRID,),
        in_specs=[_row_spec(HALF), _row_spec(HALF), _row_spec(8),
                  _full_spec(HID, OUT * HEADS), _full_spec(HID, OUT * HEADS),
                  _full_spec(1, OUT * HEADS), _full_spec(1, OUT * HEADS),
                  _full_spec(OUT * HEADS, ATT), _full_spec(1, ATT),
                  _full_spec(ATT, 8)],
        out_specs=[_row_spec(OUT * HEADS), _row_spec(OUT * HEADS),
                   _full_spec(8, 8)],
        out_shape=[jax.ShapeDtypeStruct((N, OUT * HEADS), jnp.float32),
                   jax.ShapeDtypeStruct((N, OUT * HEADS), jnp.float32),
                   jax.ShapeDtypeStruct((8, 8), jnp.float32)],
    )(hA, hB, degc, w0, w1, b0, b1, wq, bq, qv)


def _t5_body(z0_r, z1_r, beta_r, wc_r, bc_r, ho_r, lg_r):
    bv = beta_r[...]
    hout = z0_r[...] * bv[0:1, 0:1] + z1_r[...] * bv[0:1, 1:2]
    ho_r[...] = hout
    lg_r[...] = jnp.dot(hout, wc_r[...], precision=HI,
                        preferred_element_type=jnp.float32) + bc_r[...]


def _stage5(z0, z1, betav, wc, bc):
    return pl.pallas_call(
        _t5_body,
        grid=(GRID,),
        in_specs=[_row_spec(OUT * HEADS), _row_spec(OUT * HEADS),
                  _full_spec(1, 8), _full_spec(OUT * HEADS, NC),
                  _full_spec(1, NC)],
        out_specs=[_row_spec(OUT * HEADS), _row_spec(NC)],
        out_shape=[jax.ShapeDtypeStruct((N, OUT * HEADS), jnp.float32),
                   jax.ShapeDtypeStruct((N, NC), jnp.float32)],
    )(z0, z1, betav, wc, bc)


def _head_proj(a):
    return (a[:, :, None] * jnp.eye(HEADS, dtype=jnp.float32)[:, None, :]
            ).reshape(HID, HEADS)


def _gvec_from(gm):
    g = gm[0] + gm[1]
    g = jnp.where(g < 0.0, 0.2 * g, g)
    return jnp.stack([jnp.tile(g[0:4], 4), jnp.tile(g[4:8], 4)])


def _edge_tables(hd, es, ed):
    feats = jnp.stack([hd[:, :HALF], hd[:, HALF:]])
    pad4 = jnp.zeros((es.shape[0], 4), jnp.float32)
    esq = jnp.stack([jnp.concatenate([es[:, 0:4], pad4], axis=1),
                     jnp.concatenate([es[:, 4:8], pad4], axis=1)])
    edq = jnp.stack([jnp.concatenate([ed[:, 0:4], pad4], axis=1),
                     jnp.concatenate([ed[:, 4:8], pad4], axis=1)])
    return feats, esq, edq


@jax.jit
def kernel(feat0, params, adj, type_mask):
    del type_mask
    srcs = adj[0]
    dsts = adj[1]
    expand = jnp.repeat(jnp.eye(HEADS, dtype=jnp.float32), DH, axis=1)
    z128 = jnp.zeros((NP, HALF), jnp.float32)
    z16 = jnp.zeros((NP, EXW), jnp.float32)

    as1 = _head_proj(params['gat1_as'])
    ad1 = _head_proj(params['gat1_ad'])
    as2 = _head_proj(params['gat2_as'])
    ad2 = _head_proj(params['gat2_ad'])

    fs, hd1, es1, ed1, gm1 = _stage1(
        feat0, params['W_fc'], params['b_fc'][None, :],
        params['gat1_W'], as1, ad1)

    f1, esq1, edq1 = _edge_tables(hd1, es1, ed1)
    accO1, denO1 = _gat_edge(f1, esq1, edq1, srcs, dsts,
                             _gvec_from(gm1), z128, z16)

    hd2, es2, ed2, gm2 = _stage2(accO1[0, :N], accO1[1, :N],
                                 denO1[0, :N], denO1[1, :N],
                                 expand, params['gat2_W'], as2, ad2)

    f2, esq2, edq2 = _edge_tables(hd2, es2, ed2)
    accO2, denO2 = _gat_edge(f2, esq2, edq2, srcs, dsts,
                             _gvec_from(gm2), z128, z16)

    hA, hB, degc = _stage3(accO2[0, :N], accO2[1, :N],
                           denO2[0, :N], denO2[1, :N], expand, fs)

    aggO = _agg_edge(jnp.stack([hA, hB]), srcs, dsts, z128)

    z0, z1, ss = _stage4(aggO[0, :N], aggO[1, :N], degc,
                         params['W_mp'][0], params['W_mp'][1],
                         params['b_mp'][0][None, :], params['b_mp'][1][None, :],
                         params['W_q'], params['b_q'][None, :],
                         jnp.concatenate(
                             [params['q'][:, None],
                              jnp.zeros((ATT, 7), jnp.float32)], axis=1))

    beta = jax.nn.softmax(ss[0:2, 0] / N)
    betav = jnp.concatenate([beta, jnp.zeros((6,), jnp.float32)])[None, :]

    h_out, logits = _stage5(z0, z1, betav, params['W_cls'],
                            params['b_cls'][None, :])
    return (logits, h_out)
